# Initial kernel scaffold; baseline (speedup 1.0000x reference)
import jax
import jax.numpy as jnp
from jax.experimental import pallas as pl


def kernel(x, adj_matrix, W1, b1, W2, b2, alpha):
    raise NotImplementedError("write your pallas kernel here")


# speedup vs baseline: 8766.9331x; 8766.9331x over previous
import jax
import jax.numpy as jnp
from jax.experimental import pallas as pl
from jax.experimental.pallas import tpu as pltpu


def _gcn_body(x_ref, adj_ref, w1_ref, b1_ref, w2_ref, b2_ref, alpha_ref, out_ref):
    adj = adj_ref[...]
    n = adj.shape[0]
    rows = jax.lax.broadcasted_iota(jnp.int32, adj.shape, 0)
    cols = jax.lax.broadcasted_iota(jnp.int32, adj.shape, 1)
    ab = jnp.where(rows == cols, jnp.float32(1.0),
                   (adj != 0).astype(jnp.float32)).astype(jnp.bfloat16)
    at = ab.T

    deg = jnp.sum(at.astype(jnp.float32), axis=1, keepdims=True)
    s = jax.lax.rsqrt(deg)

    x = x_ref[...]
    h0 = jnp.dot(x.astype(jnp.bfloat16), w1_ref[...].astype(jnp.bfloat16),
                 preferred_element_type=jnp.float32)
    y1 = (s * h0).astype(jnp.bfloat16)
    c1 = jnp.dot(at, y1, preferred_element_type=jnp.float32)
    h1 = jax.nn.relu(s * c1 + b1_ref[...])

    g = jnp.dot(h1.astype(jnp.bfloat16), w2_ref[...].astype(jnp.bfloat16),
                preferred_element_type=jnp.float32)
    y2 = (s * g).astype(jnp.bfloat16)
    c2 = jnp.dot(at, y2, preferred_element_type=jnp.float32)
    h2 = s * c2 + b2_ref[...]

    out_ref[...] = x + alpha_ref[0, 0] * h2


def kernel(x, adj_matrix, W1, b1, W2, b2, alpha):
    n, in_dim = x.shape
    hid = W1.shape[1]
    call = pl.pallas_call(
        _gcn_body,
        out_shape=jax.ShapeDtypeStruct((n, in_dim), jnp.float32),
        compiler_params=pltpu.CompilerParams(
            vmem_limit_bytes=100 * 1024 * 1024,
        ),
    )
    return call(x, adj_matrix, W1, b1.reshape(1, hid), W2,
                b2.reshape(1, in_dim), jnp.asarray(alpha).reshape(1, 1))
